# baseline (speedup 1.0000x reference)
import functools

import jax
import jax.numpy as jnp
from jax import lax
from jax.experimental import pallas as pl
from jax.experimental.pallas import tpu as pltpu
from jax.experimental.pallas import tpu_sc as plsc

B, C, T = 16, 8, 2048
QUANT_LEVELS, QUANT_EMB = 1024, 64
NUM_CLASSES, CLASS_EMB = 1000, 128

NW = 32
ROWS_OUT = B * T * C
ROWS_COND = B * T
OUT_PER_W = ROWS_OUT // NW
COND_PER_W = ROWS_COND // NW
OUT_CHUNK = 1024
L = 16

_mesh = plsc.VectorSubcoreMesh(core_axis_name="c", subcore_axis_name="s")


@functools.partial(
    pl.kernel,
    mesh=_mesh,
    out_type=jax.ShapeDtypeStruct((ROWS_OUT, QUANT_EMB), jnp.float32),
    scratch_types=[
        pltpu.VMEM((OUT_CHUNK // 128, 128), jnp.int32),
        pltpu.VMEM((OUT_CHUNK, QUANT_EMB), jnp.float32),
        pltpu.SemaphoreType.DMA,
    ],
    compiler_params=pltpu.CompilerParams(use_tc_tiling_on_sc=False),
)
def _sc_gather_out(xp_hbm, qtab_hbm, out_hbm, idx_v, rows_v, sem):
    wid = lax.axis_index("s") * 2 + lax.axis_index("c")
    pattern = (lax.iota(jnp.int32, L) % C) * QUANT_LEVELS

    def out_step(j, _):
        base8 = wid * (OUT_PER_W // 128) + j * (OUT_CHUNK // 128)
        pltpu.sync_copy(xp_hbm.at[pl.ds(base8, OUT_CHUNK // 128)], idx_v)
        for r in range(OUT_CHUNK // 128):
            for g in range(128 // L):
                sl = pl.ds(g * L, L)
                idx_v[r, sl] = idx_v[r, sl] + pattern
        cps = [
            pltpu.async_copy(qtab_hbm.at[idx_v.at[r]],
                             rows_v.at[pl.ds(r * 128, 128)], sem)
            for r in range(OUT_CHUNK // 128)
        ]
        for cp in cps:
            cp.wait()
        pltpu.sync_copy(rows_v, out_hbm.at[pl.ds(base8 * 128, OUT_CHUNK)])
        return 0

    lax.fori_loop(0, OUT_PER_W // OUT_CHUNK, out_step, 0)


@functools.partial(
    pl.kernel,
    mesh=_mesh,
    out_type=jax.ShapeDtypeStruct((ROWS_COND, CLASS_EMB), jnp.float32),
    scratch_types=[
        pltpu.VMEM((COND_PER_W // 128, 128), jnp.int32),
        pltpu.VMEM((128, CLASS_EMB), jnp.float32),
        pltpu.SemaphoreType.DMA,
    ],
    compiler_params=pltpu.CompilerParams(use_tc_tiling_on_sc=False),
)
def _sc_gather_cond(cidx_hbm, wtab_hbm, crows_hbm, cidx_v, crows_v, sem):
    wid = lax.axis_index("s") * 2 + lax.axis_index("c")
    pltpu.sync_copy(cidx_hbm.at[pl.ds(wid * (COND_PER_W // 128),
                                      COND_PER_W // 128)], cidx_v)
    for r in range(COND_PER_W // 128):
        pltpu.async_copy(wtab_hbm.at[cidx_v.at[r]], crows_v, sem).wait()
        pltpu.sync_copy(crows_v,
                        crows_hbm.at[pl.ds(wid * COND_PER_W + r * 128, 128)])


TB = 512


def _tc_expand_body(crows_ref, cond_ref, out_ref):
    rows = crows_ref[0]
    mask = (cond_ref[0] > 0).astype(jnp.float32)
    val = rows.T * mask
    for c in range(C):
        out_ref[c] = val


def _tc_expand(crows3, condition):
    return pl.pallas_call(
        _tc_expand_body,
        grid=(B, T // TB),
        in_specs=[
            pl.BlockSpec((1, TB, CLASS_EMB), lambda b, t: (b, t, 0)),
            pl.BlockSpec((1, 1, TB), lambda b, t: (b, 0, t)),
        ],
        out_specs=pl.BlockSpec((C, CLASS_EMB, TB), lambda b, t: (b, 0, t)),
        out_shape=jax.ShapeDtypeStruct((B * C, CLASS_EMB, T), jnp.float32),
    )(crows3, condition)


IB = 1024


def _tc_unpack_body(rows2_ref, out_ref):
    blk = rows2_ref[...].reshape(IB, C // 2, 2 * QUANT_EMB)
    for c in range(C):
        out_ref[:, c, :] = blk[:, c // 2,
                               (c % 2) * QUANT_EMB:(c % 2 + 1) * QUANT_EMB]


def _tc_unpack(rows2):
    return pl.pallas_call(
        _tc_unpack_body,
        grid=(B * T // IB,),
        in_specs=[pl.BlockSpec((IB * C // 2, 128), lambda i: (i, 0))],
        out_specs=pl.BlockSpec((IB, C, QUANT_EMB), lambda i: (i, 0, 0)),
        out_shape=jax.ShapeDtypeStruct((B * T, C, QUANT_EMB), jnp.float32),
    )(rows2)


def kernel(x, condition, quant_emb, cond_emb_weight):
    xp = jnp.transpose(x, (0, 2, 1)).reshape(ROWS_OUT // 128, 128)
    cidx = condition.reshape(ROWS_COND // 128, 128)
    qtab = quant_emb.reshape(C * QUANT_LEVELS, QUANT_EMB)

    crows = _sc_gather_cond(cidx, cond_emb_weight)
    out_rows = _sc_gather_out(xp, qtab)

    cond = _tc_expand(crows.reshape(B, T, CLASS_EMB), condition)
    out = _tc_unpack(out_rows.reshape(ROWS_OUT // 2, 2 * QUANT_EMB))
    return out, cond


# speedup vs baseline: 1.6614x; 1.6614x over previous
import functools

import jax
import jax.numpy as jnp
from jax import lax
from jax.experimental import pallas as pl
from jax.experimental.pallas import tpu as pltpu
from jax.experimental.pallas import tpu_sc as plsc

B, C, T = 16, 8, 2048
QUANT_LEVELS, QUANT_EMB = 1024, 64
NUM_CLASSES, CLASS_EMB = 1000, 128

NW = 32
ROWS_OUT = B * T * C
ROWS_COND = B * T
OUT_PER_W = ROWS_OUT // NW
COND_PER_W = ROWS_COND // NW
OUT_CHUNK = 1024
L = 16

_mesh = plsc.VectorSubcoreMesh(core_axis_name="c", subcore_axis_name="s")


@functools.partial(
    pl.kernel,
    mesh=_mesh,
    out_type=jax.ShapeDtypeStruct((ROWS_OUT, QUANT_EMB), jnp.float32),
    scratch_types=[
        pltpu.VMEM((OUT_CHUNK // 128, 128), jnp.int32),
        pltpu.VMEM((OUT_CHUNK, QUANT_EMB), jnp.float32),
        pltpu.SemaphoreType.DMA,
    ],
    compiler_params=pltpu.CompilerParams(use_tc_tiling_on_sc=False),
)
def _sc_gather_out(xp_hbm, qtab_hbm, out_hbm, idx_v, rows_v, sem):
    wid = lax.axis_index("s") * 2 + lax.axis_index("c")
    pattern = (lax.iota(jnp.int32, L) % C) * QUANT_LEVELS

    def out_step(j, _):
        base8 = wid * (OUT_PER_W // 128) + j * (OUT_CHUNK // 128)
        pltpu.sync_copy(xp_hbm.at[pl.ds(base8, OUT_CHUNK // 128)], idx_v)
        for r in range(OUT_CHUNK // 128):
            for g in range(128 // L):
                sl = pl.ds(g * L, L)
                idx_v[r, sl] = idx_v[r, sl] + pattern
        cps = [
            pltpu.async_copy(qtab_hbm.at[idx_v.at[r]],
                             rows_v.at[pl.ds(r * 128, 128)], sem)
            for r in range(OUT_CHUNK // 128)
        ]
        for cp in cps:
            cp.wait()
        pltpu.sync_copy(rows_v, out_hbm.at[pl.ds(base8 * 128, OUT_CHUNK)])
        return 0

    lax.fori_loop(0, OUT_PER_W // OUT_CHUNK, out_step, 0)


@functools.partial(
    pl.kernel,
    mesh=_mesh,
    out_type=jax.ShapeDtypeStruct((ROWS_COND, CLASS_EMB), jnp.float32),
    scratch_types=[
        pltpu.VMEM((COND_PER_W // 128, 128), jnp.int32),
        pltpu.VMEM((128, CLASS_EMB), jnp.float32),
        pltpu.SemaphoreType.DMA,
    ],
    compiler_params=pltpu.CompilerParams(use_tc_tiling_on_sc=False),
)
def _sc_gather_cond(cidx_hbm, wtab_hbm, crows_hbm, cidx_v, crows_v, sem):
    wid = lax.axis_index("s") * 2 + lax.axis_index("c")
    pltpu.sync_copy(cidx_hbm.at[pl.ds(wid * (COND_PER_W // 128),
                                      COND_PER_W // 128)], cidx_v)
    for r in range(COND_PER_W // 128):
        pltpu.async_copy(wtab_hbm.at[cidx_v.at[r]], crows_v, sem).wait()
        pltpu.sync_copy(crows_v,
                        crows_hbm.at[pl.ds(wid * COND_PER_W + r * 128, 128)])


TB = 512


def _tc_expand_body(crows_ref, cond_ref, out_ref):
    rows = crows_ref[0]
    mask = (cond_ref[0] > 0).astype(jnp.float32)
    val = rows.T * mask
    for c in range(C):
        out_ref[c] = val


def _tc_expand(crows3, condition):
    return pl.pallas_call(
        _tc_expand_body,
        grid=(B, T // TB),
        in_specs=[
            pl.BlockSpec((1, TB, CLASS_EMB), lambda b, t: (b, t, 0)),
            pl.BlockSpec((1, 1, TB), lambda b, t: (b, 0, t)),
        ],
        out_specs=pl.BlockSpec((C, CLASS_EMB, TB), lambda b, t: (b, 0, t)),
        out_shape=jax.ShapeDtypeStruct((B * C, CLASS_EMB, T), jnp.float32),
    )(crows3, condition)


IBL = 2048


def _tc_trans_body(rows2_ref, out_ref):
    blk = rows2_ref[...].reshape(IBL, C // 2, 2 * QUANT_EMB)
    for c in range(C):
        sub = blk[:, c // 2,
                  (c % 2) * QUANT_EMB:(c % 2 + 1) * QUANT_EMB]
        out_ref[c] = sub.T


def _tc_trans(rows2):
    return pl.pallas_call(
        _tc_trans_body,
        grid=(B * T // IBL,),
        in_specs=[pl.BlockSpec((IBL * C // 2, 128), lambda i: (i, 0))],
        out_specs=pl.BlockSpec((C, QUANT_EMB, IBL), lambda i: (0, 0, i)),
        out_shape=jax.ShapeDtypeStruct((C, QUANT_EMB, B * T), jnp.float32),
    )(rows2)


def kernel(x, condition, quant_emb, cond_emb_weight):
    xp = jnp.transpose(x, (0, 2, 1)).reshape(ROWS_OUT // 128, 128)
    cidx = condition.reshape(ROWS_COND // 128, 128)
    qtab = quant_emb.reshape(C * QUANT_LEVELS, QUANT_EMB)

    crows = _sc_gather_cond(cidx, cond_emb_weight)
    out_rows = _sc_gather_out(xp, qtab)

    cond = _tc_expand(crows.reshape(B, T, CLASS_EMB), condition)
    out_t = _tc_trans(out_rows.reshape(ROWS_OUT // 2, 2 * QUANT_EMB))
    return jnp.transpose(out_t, (2, 0, 1)), cond
